# Initial kernel scaffold; baseline (speedup 1.0000x reference)
import jax
import jax.numpy as jnp
from jax.experimental import pallas as pl


def kernel(input, edge_index, edge_weight, weight):
    raise NotImplementedError("write your pallas kernel here")


# speedup vs baseline: 5.3434x; 5.3434x over previous
import functools

import jax
import jax.numpy as jnp
from jax import lax
from jax.experimental import pallas as pl
from jax.experimental.pallas import tpu as pltpu
from jax.experimental.pallas import tpu_sc as plsc

N_NODES = 10000
N_FEAT = 128
N_EDGES = 320000

NUM_CORES = 2
NUM_SUBCORES = 16
NUM_WORKERS = NUM_CORES * NUM_SUBCORES
CHUNK = 128
EDGES_PAD = -(-N_EDGES // (NUM_WORKERS * CHUNK)) * (NUM_WORKERS * CHUNK)
EDGES_PER_WORKER = EDGES_PAD // NUM_WORKERS
CHUNKS_PER_WORKER = EDGES_PER_WORKER // CHUNK
N_PAD = 10240
ROWS_PER_SUBCORE = N_PAD // NUM_SUBCORES
VREGS_PER_ROW = N_FEAT // 16


def _sc_aggregate(x, col, row, w, zeros):
    mesh = plsc.VectorSubcoreMesh(core_axis_name="c", subcore_axis_name="s")

    @functools.partial(
        pl.kernel,
        out_type=jax.ShapeDtypeStruct((NUM_CORES * N_PAD, N_FEAT),
                                      jnp.float32),
        mesh=mesh,
        scratch_types=[
            pltpu.VMEM((CHUNK,), jnp.int32),
            pltpu.VMEM((CHUNK,), jnp.int32),
            pltpu.VMEM((CHUNK,), jnp.float32),
            pltpu.VMEM((CHUNK, N_FEAT), jnp.float32),
            pltpu.VMEM_SHARED((N_PAD, N_FEAT), jnp.float32),
            pltpu.SemaphoreType.DMA,
        ],
    )
    def body(x_hbm, col_hbm, row_hbm, w_hbm, z_hbm, out_hbm,
             cidx_v, ridx_v, w_v, rows_v, acc_sh, sem):
        c = lax.axis_index("c")
        s = lax.axis_index("s")
        wid = c * NUM_SUBCORES + s

        srow = s * ROWS_PER_SUBCORE
        pltpu.sync_copy(z_hbm.at[pl.ds(srow, ROWS_PER_SUBCORE)],
                        acc_sh.at[pl.ds(srow, ROWS_PER_SUBCORE)])
        plsc.subcore_barrier()

        base = wid * EDGES_PER_WORKER

        def chunk_body(k, carry):
            off = base + k * CHUNK
            pltpu.sync_copy(col_hbm.at[pl.ds(off, CHUNK)], cidx_v)
            pltpu.sync_copy(row_hbm.at[pl.ds(off, CHUNK)], ridx_v)
            pltpu.sync_copy(w_hbm.at[pl.ds(off, CHUNK)], w_v)
            pltpu.async_copy(x_hbm.at[cidx_v], rows_v, sem).wait()

            def scale_body(g, carry2):
                wv = w_v[pl.ds(g * 16, 16)]
                for i in range(16):
                    r = g * 16 + i
                    sp = jnp.broadcast_to(wv[i], (16,))
                    for j in range(VREGS_PER_ROW):
                        sl = pl.ds(j * 16, 16)
                        rows_v[r, sl] = rows_v[r, sl] * sp
                return carry2

            lax.fori_loop(0, CHUNK // 16, scale_body, 0)
            pltpu.sync_copy(rows_v, acc_sh.at[ridx_v], add=True)
            return carry

        lax.fori_loop(0, CHUNKS_PER_WORKER, chunk_body, 0)

        plsc.subcore_barrier()
        pltpu.sync_copy(acc_sh.at[pl.ds(srow, ROWS_PER_SUBCORE)],
                        out_hbm.at[pl.ds(c * N_PAD + srow,
                                         ROWS_PER_SUBCORE)])

    return body(x, col, row, w, zeros)


def _tc_project(partial, weight):
    blk = 1024
    grid = N_PAD // blk

    def body(p0_ref, p1_ref, w_ref, o_ref):
        s = p0_ref[...] + p1_ref[...]
        o_ref[...] = jnp.dot(s, w_ref[...],
                             preferred_element_type=jnp.float32)

    return pl.pallas_call(
        body,
        grid=(grid,),
        in_specs=[
            pl.BlockSpec((blk, N_FEAT), lambda i: (i, 0)),
            pl.BlockSpec((blk, N_FEAT), lambda i: (i + grid, 0)),
            pl.BlockSpec((N_FEAT, N_FEAT), lambda i: (0, 0)),
        ],
        out_specs=pl.BlockSpec((blk, N_FEAT), lambda i: (i, 0)),
        out_shape=jax.ShapeDtypeStruct((N_PAD, N_FEAT), jnp.float32),
    )(partial, partial, weight)


def kernel(input, edge_index, edge_weight, weight):
    x = input.astype(jnp.float32)
    row = edge_index[0].astype(jnp.int32)
    col = edge_index[1].astype(jnp.int32)
    w = edge_weight.astype(jnp.float32)

    pad = EDGES_PAD - N_EDGES
    if pad:
        pad_idx = jnp.arange(pad, dtype=jnp.int32) % N_NODES
        row = jnp.concatenate([row, pad_idx])
        col = jnp.concatenate([col, pad_idx])
        w = jnp.concatenate([w, jnp.zeros((pad,), jnp.float32)])

    zeros = jnp.zeros((N_PAD, N_FEAT), jnp.float32)
    partial = _sc_aggregate(x, col, row, w, zeros)
    out = _tc_project(partial, weight)
    return out[:N_NODES]


# speedup vs baseline: 10.8020x; 2.0216x over previous
import functools

import jax
import jax.numpy as jnp
from jax import lax
from jax.experimental import pallas as pl
from jax.experimental.pallas import tpu as pltpu
from jax.experimental.pallas import tpu_sc as plsc

N_NODES = 10000
N_FEAT = 128
N_EDGES = 320000

NUM_CORES = 2
NUM_SUBCORES = 16
NUM_WORKERS = NUM_CORES * NUM_SUBCORES
CHUNK = 128
K_CHUNKS = 80
B_CHUNKS = 40
N_BLOCKS = K_CHUNKS // B_CHUNKS
EDGES_PER_WORKER = K_CHUNKS * CHUNK
EDGES_PAD = NUM_WORKERS * EDGES_PER_WORKER
N_PAD = 10240
ROWS_PER_SUBCORE = N_PAD // NUM_SUBCORES
VREGS_PER_ROW = N_FEAT // 16


def _sc_aggregate(x, col, row, w, zeros):
    mesh = plsc.VectorSubcoreMesh(core_axis_name="c", subcore_axis_name="s")

    @functools.partial(
        pl.kernel,
        out_type=jax.ShapeDtypeStruct((NUM_CORES * N_PAD, N_FEAT),
                                      jnp.float32),
        mesh=mesh,
        scratch_types=[
            pltpu.VMEM((B_CHUNKS, CHUNK), jnp.int32),
            pltpu.VMEM((B_CHUNKS, CHUNK), jnp.int32),
            pltpu.VMEM((B_CHUNKS, CHUNK), jnp.float32),
            pltpu.VMEM((CHUNK, N_FEAT), jnp.float32),
            pltpu.VMEM((CHUNK, N_FEAT), jnp.float32),
            pltpu.VMEM_SHARED((N_PAD, N_FEAT), jnp.float32),
            pltpu.SemaphoreType.DMA,
            pltpu.SemaphoreType.DMA,
        ],
    )
    def body(x_hbm, col_hbm, row_hbm, w_hbm, z_hbm, out_hbm,
             col_v, row_v, w_v, rows0, rows1, acc_sh, sem0, sem1):
        c = lax.axis_index("c")
        s = lax.axis_index("s")
        wid = c * NUM_SUBCORES + s

        srow = s * ROWS_PER_SUBCORE
        pltpu.sync_copy(z_hbm.at[pl.ds(srow, ROWS_PER_SUBCORE)],
                        acc_sh.at[pl.ds(srow, ROWS_PER_SUBCORE)])
        plsc.subcore_barrier()

        bufs = (rows0, rows1)
        sems = (sem0, sem1)

        for blk in range(N_BLOCKS):
            kbase = wid * K_CHUNKS + blk * B_CHUNKS
            pltpu.sync_copy(col_hbm.at[pl.ds(kbase, B_CHUNKS)], col_v)
            pltpu.sync_copy(row_hbm.at[pl.ds(kbase, B_CHUNKS)], row_v)
            pltpu.sync_copy(w_hbm.at[pl.ds(kbase, B_CHUNKS)], w_v)
            pltpu.async_copy(x_hbm.at[col_v.at[0]], rows0, sem0)

            def outer(k2, carry):
                for b in range(2):
                    k = k2 * 2 + b
                    buf, gsem = bufs[b], sems[b]
                    nbuf, ngsem = bufs[1 - b], sems[1 - b]
                    pltpu.make_async_copy(x_hbm.at[col_v.at[k]], buf,
                                          gsem).wait()

                    @pl.when(k + 1 < B_CHUNKS)
                    def _():
                        pltpu.async_copy(x_hbm.at[col_v.at[k + 1]], nbuf,
                                         ngsem)

                    def scale_body(g, carry2):
                        wv = w_v[k, pl.ds(g * 16, 16)]
                        for i in range(16):
                            r = g * 16 + i
                            sp = jnp.broadcast_to(wv[i], (16,))
                            for j in range(VREGS_PER_ROW):
                                sl = pl.ds(j * 16, 16)
                                buf[r, sl] = buf[r, sl] * sp
                        return carry2

                    lax.fori_loop(0, CHUNK // 16, scale_body, 0)
                    pltpu.sync_copy(buf, acc_sh.at[row_v.at[k]], add=True)
                return carry

            lax.fori_loop(0, B_CHUNKS // 2, outer, 0)

        plsc.subcore_barrier()
        pltpu.sync_copy(acc_sh.at[pl.ds(srow, ROWS_PER_SUBCORE)],
                        out_hbm.at[pl.ds(c * N_PAD + srow,
                                         ROWS_PER_SUBCORE)])

    return body(x, col, row, w, zeros)


def _tc_project(partial, weight):
    blk = 1024
    grid = N_PAD // blk

    def body(p0_ref, p1_ref, w_ref, o_ref):
        s = p0_ref[...] + p1_ref[...]
        o_ref[...] = jnp.dot(s, w_ref[...],
                             preferred_element_type=jnp.float32)

    return pl.pallas_call(
        body,
        grid=(grid,),
        in_specs=[
            pl.BlockSpec((blk, N_FEAT), lambda i: (i, 0)),
            pl.BlockSpec((blk, N_FEAT), lambda i: (i + grid, 0)),
            pl.BlockSpec((N_FEAT, N_FEAT), lambda i: (0, 0)),
        ],
        out_specs=pl.BlockSpec((blk, N_FEAT), lambda i: (i, 0)),
        out_shape=jax.ShapeDtypeStruct((N_PAD, N_FEAT), jnp.float32),
    )(partial, partial, weight)


def kernel(input, edge_index, edge_weight, weight):
    x = input.astype(jnp.float32)
    row = edge_index[0].astype(jnp.int32)
    col = edge_index[1].astype(jnp.int32)
    w = edge_weight.astype(jnp.float32)

    pad = EDGES_PAD - N_EDGES
    pad_idx = jnp.arange(pad, dtype=jnp.int32) % N_NODES
    row = jnp.concatenate([row, pad_idx]).reshape(-1, CHUNK)
    col = jnp.concatenate([col, pad_idx]).reshape(-1, CHUNK)
    w = jnp.concatenate([w, jnp.zeros((pad,), jnp.float32)]).reshape(-1, CHUNK)

    zeros = jnp.zeros((N_PAD, N_FEAT), jnp.float32)
    partial = _sc_aggregate(x, col, row, w, zeros)
    out = _tc_project(partial, weight)
    return out[:N_NODES]
